# Initial kernel scaffold; baseline (speedup 1.0000x reference)
import jax
import jax.numpy as jnp
from jax.experimental import pallas as pl


def kernel(x, edge_index, batch, W1, b1, W2, b2, W3, b3, W4, b4, w5, b5, w6, b6, fc1_W, fc1_b, fc2_W, fc2_b):
    raise NotImplementedError("write your pallas kernel here")


# speedup vs baseline: 4.2240x; 4.2240x over previous
import functools

import jax
import jax.numpy as jnp
from jax import lax
from jax.experimental import pallas as pl
from jax.experimental.pallas import tpu as pltpu
from jax.experimental.pallas import tpu_sc as plsc

N = 10000
NP = 10240
E = 320000
G = 128
K = 30
NUM_CLASSES = 2
NC = 2
NS = 16
NW = NC * NS
EPW = E // NW
C = 80
DUMMY = NP - 8
RPT = NP // NS

_f32 = jnp.float32


def _make_sc_scatter(F):
  mesh = plsc.VectorSubcoreMesh(
      core_axis_name="c", subcore_axis_name="s", num_cores=NC,
      num_subcores=NS)

  @functools.partial(
      pl.kernel,
      out_type=jax.ShapeDtypeStruct((NC, NP, F), _f32),
      mesh=mesh,
      compiler_params=pltpu.CompilerParams(use_tc_tiling_on_sc=False),
      scratch_types=[
          pltpu.VMEM((C,), jnp.int32),
          pltpu.VMEM((C,), jnp.int32),
          pltpu.VMEM((C, F), _f32),
          pltpu.VMEM_SHARED((NP, F), _f32),
      ],
  )
  def k(tab_hbm, src_hbm, dst_hbm, zeros_hbm, out_hbm, sidx, didx, rows, acc):
    c = lax.axis_index("c")
    s = lax.axis_index("s")
    wid = s * NC + c
    r0 = pl.multiple_of(s * RPT, 8)
    pltpu.sync_copy(zeros_hbm.at[pl.ds(r0, RPT)], acc.at[pl.ds(r0, RPT)])
    plsc.subcore_barrier()
    base = wid * EPW

    def body(i, carry):
      off = pl.multiple_of(base + i * C, 8)
      pltpu.sync_copy(src_hbm.at[pl.ds(off, C)], sidx)
      pltpu.sync_copy(dst_hbm.at[pl.ds(off, C)], didx)
      pltpu.sync_copy(tab_hbm.at[sidx], rows)
      pltpu.sync_copy(rows, acc.at[didx], add=True)
      return carry

    lax.fori_loop(0, EPW // C, body, 0)
    plsc.subcore_barrier()
    pltpu.sync_copy(acc.at[pl.ds(r0, RPT)], out_hbm.at[c, pl.ds(r0, RPT)])

  return k


_sc32 = _make_sc_scatter(32)
_sc16 = _make_sc_scatter(16)


def _tc_pre(x, W1, degP):
  def body(x_ref, w_ref, dp_ref, g_ref, dv_ref):
    deg = dp_ref[0, :, 0:1] + dp_ref[1, :, 0:1] + 1.0
    dinv = lax.rsqrt(deg)
    dinv32 = jnp.broadcast_to(dinv, (NP, 32))
    dv_ref[...] = dinv32
    h = jnp.dot(x_ref[...], w_ref[...], preferred_element_type=_f32)
    g_ref[...] = dinv32 * h

  return pl.pallas_call(
      body,
      out_shape=(jax.ShapeDtypeStruct((NP, 32), _f32),
                 jax.ShapeDtypeStruct((NP, 32), _f32)),
  )(x, W1, degP)


def _tc_mid(P, Gcur, dinv32, b_row, Wn, Fn):
  def body(p_ref, g_ref, d_ref, b_ref, w_ref, xo_ref, gn_ref):
    acc = p_ref[0] + p_ref[1]
    xo = jnp.tanh(d_ref[...] * (acc + g_ref[...]) + b_ref[...])
    xo_ref[...] = xo
    h = jnp.dot(xo, w_ref[...], preferred_element_type=_f32)
    gn_ref[...] = d_ref[:, 0:Fn] * h

  return pl.pallas_call(
      body,
      out_shape=(jax.ShapeDtypeStruct((NP, 32), _f32),
                 jax.ShapeDtypeStruct((NP, Fn), _f32)),
  )(P, Gcur, dinv32, b_row, Wn)


def _tc_post(P, Gcur, dinv32, b_row):
  def body(p_ref, g_ref, d_ref, b_ref, xo_ref):
    acc = p_ref[0] + p_ref[1]
    xo_ref[...] = jnp.tanh(d_ref[:, 0:16] * (acc + g_ref[...]) + b_ref[...])

  return pl.pallas_call(
      body, out_shape=jax.ShapeDtypeStruct((NP, 16), _f32),
  )(P, Gcur, dinv32, b_row)


def _pool_head(xc, lastc, batchr, W5p, b5r, W6s, b6r, fc1r, f1br, fc2p, f2br):
  def body(xc_ref, lc_ref, bt_ref, w5_ref, b5_ref, w6_ref, b6_ref,
           f1_ref, f1b_ref, f2_ref, f2b_ref, out_ref):
    g = pl.program_id(0)
    neg = _f32(-1e30)
    mask = bt_ref[...] == g
    vals = jnp.where(mask, lc_ref[...], neg)
    cnt = jnp.sum(mask.astype(_f32))
    flat = lax.broadcasted_iota(jnp.int32, (1, NP), 1)
    rows = []
    for kk in range(K):
      mval = jnp.max(vals)
      eq = vals == mval
      m = jnp.min(jnp.where(eq, flat, jnp.int32(NP)))
      sel = flat == m
      onehot = sel.astype(_f32)
      row = jnp.dot(onehot, xc_ref[...], preferred_element_type=_f32)
      valid = (_f32(kk) < cnt).astype(_f32)
      rows.append(row * valid)
      vals = jnp.where(sel, neg, vals)
    Pm = jnp.concatenate(rows, axis=0)
    c1 = jnp.dot(Pm, w5_ref[...], preferred_element_type=_f32) + b5_ref[...]
    c1 = jnp.maximum(c1, 0.0)
    hp = jnp.concatenate(
        [jnp.maximum(c1[2 * t:2 * t + 1, :], c1[2 * t + 1:2 * t + 2, :])
         for t in range(K // 2)], axis=0)
    acc2 = None
    for d in range(5):
      term = jnp.dot(hp[d:d + 11, :], w6_ref[d * 16:(d + 1) * 16, :],
                     preferred_element_type=_f32)
      acc2 = term if acc2 is None else acc2 + term
    h2 = jnp.maximum(acc2 + b6_ref[...], 0.0)
    accf = None
    for t in range(11):
      term = jnp.dot(h2[t:t + 1, :], f1_ref[t * 32:(t + 1) * 32, :],
                     preferred_element_type=_f32)
      accf = term if accf is None else accf + term
    v = jnp.maximum(accf + f1b_ref[...], 0.0)
    logits = jnp.dot(v, f2_ref[...], preferred_element_type=_f32) + f2b_ref[...]
    lane = lax.broadcasted_iota(jnp.int32, (1, 128), 1)
    lmask = lane < NUM_CLASSES
    mx = jnp.max(jnp.where(lmask, logits, neg))
    ssum = jnp.sum(jnp.where(lmask, jnp.exp(logits - mx), 0.0))
    out_ref[0] = logits - mx - jnp.log(ssum)

  cst = lambda shape: pl.BlockSpec(shape, lambda g: (0,) * len(shape))
  return pl.pallas_call(
      body,
      grid=(G,),
      in_specs=[
          cst((NP, 128)), cst((1, NP)), cst((1, NP)), cst((128, 16)),
          cst((1, 16)), cst((80, 32)), cst((1, 32)), cst((352, 128)),
          cst((1, 128)), cst((128, 128)), cst((1, 128)),
      ],
      out_specs=pl.BlockSpec((1, 1, 128), lambda g: (g, 0, 0)),
      out_shape=jax.ShapeDtypeStruct((G, 1, 128), _f32),
  )(xc, lastc, batchr, W5p, b5r, W6s, b6r, fc1r, f1br, fc2p, f2br)


def kernel(x, edge_index, batch, W1, b1, W2, b2, W3, b3, W4, b4,
           w5, b5, w6, b6, fc1_W, fc1_b, fc2_W, fc2_b):
  src = edge_index[0].astype(jnp.int32)
  dst = edge_index[1].astype(jnp.int32)
  dstf = jnp.where(src == dst, jnp.int32(DUMMY), dst)

  xpad = jnp.pad(x, ((0, NP - N), (0, 0)))
  ones16 = jnp.ones((NP, 16), _f32)
  zeros16 = jnp.zeros((NP, 16), _f32)
  zeros32 = jnp.zeros((NP, 32), _f32)

  degP = _sc16(ones16, src, dstf, zeros16)
  G1, dinv32 = _tc_pre(xpad, W1, degP)

  W4p = jnp.pad(W4, ((0, 0), (0, 15)))
  b4p = jnp.pad(b4, (0, 15)).reshape(1, 16)

  P1 = _sc32(G1, src, dstf, zeros32)
  x1, G2 = _tc_mid(P1, G1, dinv32, b1.reshape(1, 32), W2, 32)
  P2 = _sc32(G2, src, dstf, zeros32)
  x2, G3 = _tc_mid(P2, G2, dinv32, b2.reshape(1, 32), W3, 32)
  P3 = _sc32(G3, src, dstf, zeros32)
  x3, G4 = _tc_mid(P3, G3, dinv32, b3.reshape(1, 32), W4p, 16)
  P4 = _sc16(G4, src, dstf, zeros16)
  x4 = _tc_post(P4, G4, dinv32, b4p)

  xc = jnp.concatenate(
      [x1, x2, x3, x4[:, 0:1], jnp.zeros((NP, 31), _f32)], axis=1)
  lastc = x4[:, 0].reshape(1, NP)
  batchr = jnp.concatenate(
      [batch.astype(jnp.int32), jnp.full((NP - N,), -1, jnp.int32)]
  ).reshape(1, NP)

  W5p = jnp.zeros((128, 16), _f32).at[:97, :].set(w5[:, 0, :].T)
  b5r = b5.reshape(1, 16)
  W6s = w6.transpose(2, 1, 0).reshape(80, 32)
  b6r = b6.reshape(1, 32)
  fc1r = fc1_W.reshape(32, 11, 128).transpose(1, 0, 2).reshape(352, 128)
  f1br = fc1_b.reshape(1, 128)
  fc2p = jnp.zeros((128, 128), _f32).at[:, :NUM_CLASSES].set(fc2_W)
  f2br = jnp.zeros((1, 128), _f32).at[0, :NUM_CLASSES].set(fc2_b)

  out = _pool_head(xc, lastc, batchr, W5p, b5r, W6s, b6r,
                   fc1r, f1br, fc2p, f2br)
  return out.reshape(G, 128)[:, :NUM_CLASSES]


# speedup vs baseline: 5.1577x; 1.2210x over previous
import functools

import jax
import jax.numpy as jnp
from jax import lax
from jax.experimental import pallas as pl
from jax.experimental.pallas import tpu as pltpu
from jax.experimental.pallas import tpu_sc as plsc

N = 10000
NP = 10240
E = 320000
G = 128
K = 30
NUM_CLASSES = 2
NC = 2
NS = 16
NW = NC * NS
EPW = E // NW
C = 80
DUMMY = NP - 8
RPT = NP // NS

_f32 = jnp.float32


def _make_sc_scatter(F):
  mesh = plsc.VectorSubcoreMesh(
      core_axis_name="c", subcore_axis_name="s", num_cores=NC,
      num_subcores=NS)

  @functools.partial(
      pl.kernel,
      out_type=jax.ShapeDtypeStruct((NC, NP, F), _f32),
      mesh=mesh,
      compiler_params=pltpu.CompilerParams(use_tc_tiling_on_sc=False),
      scratch_types=[
          pltpu.VMEM((C,), jnp.int32),
          pltpu.VMEM((C,), jnp.int32),
          pltpu.VMEM((C, F), _f32),
          pltpu.VMEM_SHARED((NP, F), _f32),
      ],
  )
  def k(tab_hbm, src_hbm, dst_hbm, zeros_hbm, out_hbm, sidx, didx, rows, acc):
    c = lax.axis_index("c")
    s = lax.axis_index("s")
    wid = s * NC + c
    r0 = pl.multiple_of(s * RPT, 8)
    pltpu.sync_copy(zeros_hbm.at[pl.ds(r0, RPT)], acc.at[pl.ds(r0, RPT)])
    plsc.subcore_barrier()
    base = wid * EPW

    def body(i, carry):
      off = pl.multiple_of(base + i * C, 8)
      pltpu.sync_copy(src_hbm.at[pl.ds(off, C)], sidx)
      pltpu.sync_copy(dst_hbm.at[pl.ds(off, C)], didx)
      pltpu.sync_copy(tab_hbm.at[sidx], rows)
      pltpu.sync_copy(rows, acc.at[didx], add=True)
      return carry

    lax.fori_loop(0, EPW // C, body, 0)
    plsc.subcore_barrier()
    pltpu.sync_copy(acc.at[pl.ds(r0, RPT)], out_hbm.at[c, pl.ds(r0, RPT)])

  return k


_sc32 = _make_sc_scatter(32)
_sc16 = _make_sc_scatter(16)


def _tc_pre(x, W1, degP):
  def body(x_ref, w_ref, dp_ref, g_ref, dv_ref):
    deg = dp_ref[0, :, 0:1] + dp_ref[1, :, 0:1] + 1.0
    dinv = lax.rsqrt(deg)
    dinv32 = jnp.broadcast_to(dinv, (NP, 32))
    dv_ref[...] = dinv32
    h = jnp.dot(x_ref[...], w_ref[...], preferred_element_type=_f32)
    g_ref[...] = dinv32 * h

  return pl.pallas_call(
      body,
      out_shape=(jax.ShapeDtypeStruct((NP, 32), _f32),
                 jax.ShapeDtypeStruct((NP, 32), _f32)),
  )(x, W1, degP)


def _tc_mid(P, Gcur, dinv32, b_row, Wn, Fn):
  def body(p_ref, g_ref, d_ref, b_ref, w_ref, xo_ref, gn_ref):
    acc = p_ref[0] + p_ref[1]
    xo = jnp.tanh(d_ref[...] * (acc + g_ref[...]) + b_ref[...])
    xo_ref[...] = xo
    h = jnp.dot(xo, w_ref[...], preferred_element_type=_f32)
    gn_ref[...] = d_ref[:, 0:Fn] * h

  return pl.pallas_call(
      body,
      out_shape=(jax.ShapeDtypeStruct((NP, 32), _f32),
                 jax.ShapeDtypeStruct((NP, Fn), _f32)),
  )(P, Gcur, dinv32, b_row, Wn)


def _tc_post(P, Gcur, dinv32, b_row):
  def body(p_ref, g_ref, d_ref, b_ref, xo_ref):
    acc = p_ref[0] + p_ref[1]
    xo_ref[...] = jnp.tanh(d_ref[:, 0:16] * (acc + g_ref[...]) + b_ref[...])

  return pl.pallas_call(
      body, out_shape=jax.ShapeDtypeStruct((NP, 16), _f32),
  )(P, Gcur, dinv32, b_row)


def _pool_head(xc, lastc, batchr, W5p, b5r, W6s, b6r, fc1r, f1br, fc2p, f2br):
  def body(xc_ref, lc_ref, bt_ref, w5_ref, b5_ref, w6_ref, b6_ref,
           f1_ref, f1b_ref, f2_ref, f2b_ref, out_ref):
    g = pl.program_id(0)
    neg = _f32(-1e30)
    mask = bt_ref[...] == g
    vals = jnp.where(mask, lc_ref[...], neg)
    cnt = jnp.sum(mask.astype(jnp.int32))
    r8 = lax.broadcasted_iota(jnp.int32, (8, NP // 8), 0)
    c8 = lax.broadcasted_iota(jnp.int32, (8, NP // 8), 1)
    flat = r8 * (NP // 8) + c8
    ms = []
    for kk in range(K):
      mval = jnp.max(vals)
      eq = vals == mval
      m = jnp.min(jnp.where(eq, flat, jnp.int32(NP)))
      ms.append(lax.broadcast_in_dim(m, (1, 1), ()))
      vals = jnp.where(flat == m, neg, vals)
    msv = jnp.concatenate(ms, axis=0)
    rowi = lax.broadcasted_iota(jnp.int32, (K, 1), 0)
    validc = (rowi < cnt).astype(_f32)
    frow = lax.broadcasted_iota(jnp.int32, (1, NP), 1)
    onehot = (frow == msv).astype(_f32) * validc
    Pm = jnp.dot(onehot, xc_ref[...], preferred_element_type=_f32)
    c1 = jnp.dot(Pm, w5_ref[...], preferred_element_type=_f32) + b5_ref[...]
    c1 = jnp.maximum(c1, 0.0)
    hp = jnp.concatenate(
        [jnp.maximum(c1[2 * t:2 * t + 1, :], c1[2 * t + 1:2 * t + 2, :])
         for t in range(K // 2)], axis=0)
    acc2 = None
    for d in range(5):
      term = jnp.dot(hp[d:d + 11, :], w6_ref[d * 16:(d + 1) * 16, :],
                     preferred_element_type=_f32)
      acc2 = term if acc2 is None else acc2 + term
    h2 = jnp.maximum(acc2 + b6_ref[...], 0.0)
    accf = None
    for t in range(11):
      term = jnp.dot(h2[t:t + 1, :], f1_ref[t * 32:(t + 1) * 32, :],
                     preferred_element_type=_f32)
      accf = term if accf is None else accf + term
    v = jnp.maximum(accf + f1b_ref[...], 0.0)
    logits = jnp.dot(v, f2_ref[...], preferred_element_type=_f32) + f2b_ref[...]
    lane = lax.broadcasted_iota(jnp.int32, (1, 128), 1)
    lmask = lane < NUM_CLASSES
    mx = jnp.max(jnp.where(lmask, logits, neg))
    ssum = jnp.sum(jnp.where(lmask, jnp.exp(logits - mx), 0.0))
    out_ref[0] = logits - mx - jnp.log(ssum)

  cst = lambda shape: pl.BlockSpec(shape, lambda g: (0,) * len(shape))
  return pl.pallas_call(
      body,
      grid=(G,),
      in_specs=[
          cst((NP, 128)), cst((8, NP // 8)), cst((8, NP // 8)), cst((128, 16)),
          cst((1, 16)), cst((80, 32)), cst((1, 32)), cst((352, 128)),
          cst((1, 128)), cst((128, 128)), cst((1, 128)),
      ],
      out_specs=pl.BlockSpec((1, 1, 128), lambda g: (g, 0, 0)),
      out_shape=jax.ShapeDtypeStruct((G, 1, 128), _f32),
  )(xc, lastc, batchr, W5p, b5r, W6s, b6r, fc1r, f1br, fc2p, f2br)


def kernel(x, edge_index, batch, W1, b1, W2, b2, W3, b3, W4, b4,
           w5, b5, w6, b6, fc1_W, fc1_b, fc2_W, fc2_b):
  src = edge_index[0].astype(jnp.int32)
  dst = edge_index[1].astype(jnp.int32)
  dstf = jnp.where(src == dst, jnp.int32(DUMMY), dst)

  xpad = jnp.pad(x, ((0, NP - N), (0, 0)))
  ones16 = jnp.ones((NP, 16), _f32)
  zeros16 = jnp.zeros((NP, 16), _f32)
  zeros32 = jnp.zeros((NP, 32), _f32)

  degP = _sc16(ones16, src, dstf, zeros16)
  G1, dinv32 = _tc_pre(xpad, W1, degP)

  W4p = jnp.pad(W4, ((0, 0), (0, 15)))
  b4p = jnp.pad(b4, (0, 15)).reshape(1, 16)

  P1 = _sc32(G1, src, dstf, zeros32)
  x1, G2 = _tc_mid(P1, G1, dinv32, b1.reshape(1, 32), W2, 32)
  P2 = _sc32(G2, src, dstf, zeros32)
  x2, G3 = _tc_mid(P2, G2, dinv32, b2.reshape(1, 32), W3, 32)
  P3 = _sc32(G3, src, dstf, zeros32)
  x3, G4 = _tc_mid(P3, G3, dinv32, b3.reshape(1, 32), W4p, 16)
  P4 = _sc16(G4, src, dstf, zeros16)
  x4 = _tc_post(P4, G4, dinv32, b4p)

  xc = jnp.concatenate(
      [x1, x2, x3, x4[:, 0:1], jnp.zeros((NP, 31), _f32)], axis=1)
  lastc = x4[:, 0].reshape(8, NP // 8)
  batchr = jnp.concatenate(
      [batch.astype(jnp.int32), jnp.full((NP - N,), -1, jnp.int32)]
  ).reshape(8, NP // 8)

  W5p = jnp.zeros((128, 16), _f32).at[:97, :].set(w5[:, 0, :].T)
  b5r = b5.reshape(1, 16)
  W6s = w6.transpose(2, 1, 0).reshape(80, 32)
  b6r = b6.reshape(1, 32)
  fc1r = fc1_W.reshape(32, 11, 128).transpose(1, 0, 2).reshape(352, 128)
  f1br = fc1_b.reshape(1, 128)
  fc2p = jnp.zeros((128, 128), _f32).at[:, :NUM_CLASSES].set(fc2_W)
  f2br = jnp.zeros((1, 128), _f32).at[0, :NUM_CLASSES].set(fc2_b)

  out = _pool_head(xc, lastc, batchr, W5p, b5r, W6s, b6r,
                   fc1r, f1br, fc2p, f2br)
  return out.reshape(G, 128)[:, :NUM_CLASSES]
